# baseline (speedup 1.0000x reference)
import jax
import jax.numpy as jnp
from jax import lax
from jax.experimental import pallas as pl
from jax.experimental.pallas import tpu as pltpu

VOCAB = 100000
OBJ = 100000
DIM = 200
N_NEG = 100
N_IDX = N_NEG + 1
LANE = 128
BAND = 8
N_R = DIM // BAND


def _body(colsblk, colslane, word, o_ref, w_ref, res, wout, acc_num, acc_sq):
    r = pl.program_id(0)
    wr = lax.rem(word[0], BAND)
    w = w_ref[pl.ds(wr, 1), :]
    os_ = o_ref[...]

    ii = lax.broadcasted_iota(jnp.int32, (DIM, BAND), 0)
    jj = lax.broadcasted_iota(jnp.int32, (DIM, BAND), 1)
    sel = (ii == r * BAND + jj).astype(jnp.float32)
    w8 = jnp.dot(w, sel, preferred_element_type=jnp.float32,
                 precision=lax.Precision.HIGHEST)
    part = jnp.dot(w8, os_, preferred_element_type=jnp.float32,
                   precision=lax.Precision.HIGHEST)
    sqp = jnp.dot(jnp.ones((1, BAND), jnp.float32), os_ * os_,
                  preferred_element_type=jnp.float32,
                  precision=lax.Precision.HIGHEST)

    @pl.when(r == 0)
    def _():
        acc_num[...] = part
        acc_sq[...] = sqp
        wout[...] = w

    @pl.when(r > 0)
    def _():
        acc_num[...] = acc_num[...] + part
        acc_sq[...] = acc_sq[...] + sqp

    @pl.when(r == N_R - 1)
    def _():
        eps2 = jnp.float32(1e-16)
        wsq = jnp.sum(w * w)
        denom2 = jnp.maximum(wsq, eps2) * jnp.maximum(acc_sq[...], eps2)
        acc_num[...] = acc_num[...] * lax.rsqrt(denom2)

        lane_iota = lax.broadcasted_iota(jnp.int32, (1, LANE), 1)
        out = jnp.zeros((1, LANE), jnp.float32)
        for i in range(N_IDX):
            blk = acc_num[0:1, pl.ds(pl.multiple_of(colsblk[i], LANE), LANE)]
            sel = jnp.sum(jnp.where(lane_iota == colslane[i], blk, 0.0))
            out = jnp.where(lane_iota == i, sel, out)
        res[...] = out


_grid_spec = pltpu.PrefetchScalarGridSpec(
    num_scalar_prefetch=3,
    grid=(N_R,),
    in_specs=[
        pl.BlockSpec((BAND, OBJ), lambda r, cb, cl, word: (r, 0)),
        pl.BlockSpec((8, DIM), lambda r, cb, cl, word: (word[0] // 8, 0)),
    ],
    out_specs=[
        pl.BlockSpec((1, LANE), lambda r, cb, cl, word: (0, 0)),
        pl.BlockSpec((1, DIM), lambda r, cb, cl, word: (0, 0)),
    ],
    scratch_shapes=[
        pltpu.VMEM((1, OBJ), jnp.float32),
        pltpu.VMEM((1, OBJ), jnp.float32),
    ],
)

_tc_call = pl.pallas_call(
    _body,
    grid_spec=_grid_spec,
    out_shape=(
        jax.ShapeDtypeStruct((1, LANE), jnp.float32),
        jax.ShapeDtypeStruct((1, DIM), jnp.float32),
    ),
)


def kernel(W, O, word, obj, neg_samples):
    word = jnp.asarray(word, jnp.int32).reshape(1)
    obj = jnp.asarray(obj, jnp.int32)
    neg = jnp.asarray(neg_samples, jnp.int32)
    cols = jnp.concatenate([obj.reshape(1), neg])
    colsblk = (cols // LANE) * LANE
    colslane = cols % LANE

    res, wout = _tc_call(colsblk, colslane, word, O, W)
    word_embed = wout
    obj_embed = res[0, 0]
    neg_embeds = res[0, 1:1 + N_NEG]
    return (word_embed, obj_embed, neg_embeds)


# speedup vs baseline: 3.3419x; 3.3419x over previous
import jax
import jax.numpy as jnp
from jax import lax
from jax.experimental import pallas as pl
from jax.experimental.pallas import tpu as pltpu

VOCAB = 100000
OBJ = 100000
DIM = 200
N_NEG = 100
N_IDX = N_NEG + 1
LANE = 128
BAND = 8
N_R = DIM // BAND


def _body(colsblk, colslane, word, o_ref, w_ref, res, wout, acc_num, acc_sq):
    r = pl.program_id(0)
    wr = lax.rem(word[0], BAND)
    w = w_ref[pl.ds(wr, 1), :]
    os_ = o_ref[...]

    ii = lax.broadcasted_iota(jnp.int32, (DIM, BAND), 0)
    jj = lax.broadcasted_iota(jnp.int32, (DIM, BAND), 1)
    sel = (ii == r * BAND + jj).astype(jnp.float32)
    w8 = jnp.dot(w, sel, preferred_element_type=jnp.float32,
                 precision=lax.Precision.HIGHEST)
    eye = (lax.broadcasted_iota(jnp.int32, (BAND, BAND), 0)
           == lax.broadcasted_iota(jnp.int32, (BAND, BAND), 1))
    wcol8 = jnp.sum(jnp.where(eye, jnp.broadcast_to(w8, (BAND, BAND)), 0.0),
                    axis=1, keepdims=True)

    @pl.when(r == 0)
    def _():
        acc_num[...] = wcol8 * os_
        acc_sq[...] = os_ * os_
        wout[...] = w

    @pl.when(r > 0)
    def _():
        acc_num[...] = acc_num[...] + wcol8 * os_
        acc_sq[...] = acc_sq[...] + os_ * os_

    @pl.when(r == N_R - 1)
    def _():
        eps2 = jnp.float32(1e-16)
        wsq = jnp.sum(w * w)
        num = jnp.sum(acc_num[...], axis=0, keepdims=True)
        sq = jnp.sum(acc_sq[...], axis=0, keepdims=True)
        denom2 = jnp.maximum(wsq, eps2) * jnp.maximum(sq, eps2)
        rall = num * lax.rsqrt(denom2)
        acc_num[0:1, :] = rall

        lane_iota = lax.broadcasted_iota(jnp.int32, (1, LANE), 1)
        out = jnp.zeros((1, LANE), jnp.float32)
        for i in range(N_IDX):
            blk = acc_num[0:1, pl.ds(pl.multiple_of(colsblk[i], LANE), LANE)]
            pick = jnp.sum(jnp.where(lane_iota == colslane[i], blk, 0.0))
            out = jnp.where(lane_iota == i, pick, out)
        res[...] = out


_grid_spec = pltpu.PrefetchScalarGridSpec(
    num_scalar_prefetch=3,
    grid=(N_R,),
    in_specs=[
        pl.BlockSpec((BAND, OBJ), lambda r, cb, cl, word: (r, 0)),
        pl.BlockSpec((8, DIM), lambda r, cb, cl, word: (word[0] // 8, 0)),
    ],
    out_specs=[
        pl.BlockSpec((1, LANE), lambda r, cb, cl, word: (0, 0)),
        pl.BlockSpec((1, DIM), lambda r, cb, cl, word: (0, 0)),
    ],
    scratch_shapes=[
        pltpu.VMEM((BAND, OBJ), jnp.float32),
        pltpu.VMEM((BAND, OBJ), jnp.float32),
    ],
)

_tc_call = pl.pallas_call(
    _body,
    grid_spec=_grid_spec,
    out_shape=(
        jax.ShapeDtypeStruct((1, LANE), jnp.float32),
        jax.ShapeDtypeStruct((1, DIM), jnp.float32),
    ),
)


def kernel(W, O, word, obj, neg_samples):
    word = jnp.asarray(word, jnp.int32).reshape(1)
    obj = jnp.asarray(obj, jnp.int32)
    neg = jnp.asarray(neg_samples, jnp.int32)
    cols = jnp.concatenate([obj.reshape(1), neg])
    colsblk = (cols // LANE) * LANE
    colslane = cols % LANE

    res, wout = _tc_call(colsblk, colslane, word, O, W)
    word_embed = wout
    obj_embed = res[0, 0]
    neg_embeds = res[0, 1:1 + N_NEG]
    return (word_embed, obj_embed, neg_embeds)


# speedup vs baseline: 3.4587x; 1.0350x over previous
import jax
import jax.numpy as jnp
from jax import lax
from jax.experimental import pallas as pl
from jax.experimental.pallas import tpu as pltpu

VOCAB = 100000
OBJ = 100000
DIM = 200
N_NEG = 100
N_IDX = N_NEG + 1
LANE = 128
NBUF = 8


def _body(cols, word, o_any, w_any, res, wout, w_vmem, sem):
    wi = word[0]
    cp = pltpu.make_async_copy(
        w_any.at[pl.ds(pl.multiple_of((wi // 8) * 8, 8), 8), :], w_vmem, sem)
    cp.start()
    cp.wait()
    w = w_vmem[pl.ds(lax.rem(wi, 8), 1), :]
    wsq = jnp.sum(w * w)
    wout[...] = w
    res[...] = jnp.zeros((1, LANE), jnp.float32)
    lane_iota = lax.broadcasted_iota(jnp.int32, (1, LANE), 1)
    eps2 = jnp.float32(1e-16)

    def step(idx, o_blk):
        i = idx[0]
        lane = lax.rem(cols[i], LANE)
        o = o_blk[...]
        num_full = jnp.dot(w, o, preferred_element_type=jnp.float32,
                           precision=lax.Precision.HIGHEST)
        sq_full = jnp.sum(o * o, axis=0, keepdims=True)
        denom2 = jnp.maximum(wsq, eps2) * jnp.maximum(sq_full, eps2)
        r_vec = num_full * lax.rsqrt(denom2)
        r_scalar = jnp.sum(jnp.where(lane_iota == lane, r_vec, 0.0))
        res[...] = jnp.where(lane_iota == i, r_scalar, res[...])

    pltpu.emit_pipeline(
        step,
        grid=(N_IDX,),
        in_specs=[pl.BlockSpec((DIM, LANE), lambda i: (0, cols[i] // LANE),
                               pipeline_mode=pl.Buffered(buffer_count=NBUF))],
        _explicit_indices=True,
    )(o_any)


_tc_call = pl.pallas_call(
    _body,
    in_specs=[
        pl.BlockSpec(memory_space=pltpu.SMEM),
        pl.BlockSpec(memory_space=pltpu.SMEM),
        pl.BlockSpec(memory_space=pl.ANY),
        pl.BlockSpec(memory_space=pl.ANY),
    ],
    out_specs=[
        pl.BlockSpec(memory_space=pltpu.VMEM),
        pl.BlockSpec(memory_space=pltpu.VMEM),
    ],
    out_shape=(
        jax.ShapeDtypeStruct((1, LANE), jnp.float32),
        jax.ShapeDtypeStruct((1, DIM), jnp.float32),
    ),
    scratch_shapes=[
        pltpu.VMEM((8, DIM), jnp.float32),
        pltpu.SemaphoreType.DMA,
    ],
)


def kernel(W, O, word, obj, neg_samples):
    word = jnp.asarray(word, jnp.int32).reshape(1)
    obj = jnp.asarray(obj, jnp.int32)
    neg = jnp.asarray(neg_samples, jnp.int32)
    cols = jnp.concatenate([obj.reshape(1), neg])

    res, wout = _tc_call(cols, word, O, W)
    word_embed = wout
    obj_embed = res[0, 0]
    neg_embeds = res[0, 1:1 + N_NEG]
    return (word_embed, obj_embed, neg_embeds)
